# baseline (speedup 1.0000x reference)
import dataclasses
import functools

import jax
import jax.numpy as jnp
from jax import lax
from jax.experimental import pallas as pl
from jax.experimental.pallas import tpu as pltpu
from jax.experimental.pallas import tpu_sc as plsc

N_NODES = 10000
N_EDGES = 320000
D_IN = 128
D_HID = 256
D_OUT = 128

NC = 2
NS = 16
LANES = 16

CHUNK = 128
N_PAD = 10240
E_PAD = 331776

ROWS_PER_SUB = N_PAD // NS
ZROWS = 16
NSLOT = 3
EDROWS = 8


def _spmm_kernel_body(edge_split_cores, dcols, x_hbm, ed_hbm, p_hbm,
                      ed, rows, zbuf, accum, esem, gsem, ssem):
    c = lax.axis_index("c")
    s = lax.axis_index("s")
    ngroups = dcols // LANES

    @pl.loop(0, ZROWS)
    def _(i):
        for g in range(ngroups):
            zbuf[i, pl.ds(g * LANES, LANES)] = jnp.zeros((LANES,), jnp.float32)

    @pl.loop(0, ROWS_PER_SUB // ZROWS)
    def _(j):
        pltpu.sync_copy(zbuf, accum.at[pl.ds(s * ROWS_PER_SUB + j * ZROWS, ZROWS), :])

    plsc.subcore_barrier()

    if edge_split_cores:
        wid = s * NC + c
        nchunks = E_PAD // (NC * NS * CHUNK)
        row_off = None
    else:
        wid = s
        nchunks = E_PAD // (NS * CHUNK)
        row_off = c * N_NODES
    base = wid * nchunks

    def ed_start(t, be):
        pltpu.async_copy(ed_hbm.at[base + t], ed.at[be], esem.at[be])

    def ed_wait(be):
        pltpu.make_async_copy(ed_hbm.at[0], ed.at[be], esem.at[be]).wait()

    def adjust(be):
        if row_off is not None:
            for q in range(CHUNK // LANES):
                sl = (be, 0, pl.ds(q * LANES, LANES))
                ed[sl] = ed[sl] + row_off

    def gather_start(br, be):
        pltpu.async_copy(x_hbm.at[ed.at[be, 0]], rows.at[br], gsem.at[br])

    def gather_wait(br, be):
        pltpu.make_async_copy(x_hbm.at[ed.at[be, 0]], rows.at[br],
                              gsem.at[br]).wait()

    def mult(br, be):
        @pl.loop(0, CHUNK // LANES)
        def _(q):
            wv = plsc.bitcast(ed[be, 2, pl.ds(q * LANES, LANES)], jnp.float32)
            for j in range(LANES):
                wt = wv[j]
                row = q * LANES + j
                for g in range(ngroups):
                    sl = (br, row, pl.ds(g * LANES, LANES))
                    rows[sl] = rows[sl] * wt

    def scatter_start(br, be):
        pltpu.async_copy(rows.at[br], accum.at[ed.at[be, 1]], ssem.at[br],
                         add=True)

    def scatter_wait(br, be):
        pltpu.make_async_copy(rows.at[br], accum.at[ed.at[be, 1]],
                              ssem.at[br]).wait()

    def full_step(t, tr=None):
        ti = t if tr is None else tr
        br, be = t % 2, t % 3
        gather_wait(br, be)
        mult(br, be)
        scatter_start(br, be)
        scatter_wait(1 - br, (t + 2) % 3)
        ed_start(ti + 2, (t + 2) % 3)
        ed_wait((t + 1) % 3)
        adjust((t + 1) % 3)
        gather_start(1 - br, (t + 1) % 3)

    ed_start(0, 0)
    ed_start(1, 1)
    ed_wait(0)
    adjust(0)
    gather_start(0, 0)
    gather_wait(0, 0)
    mult(0, 0)
    scatter_start(0, 0)
    ed_start(2, 2)
    ed_wait(1)
    adjust(1)
    gather_start(1, 1)

    n_steady = ((nchunks - 3) // 6) * 6

    @pl.loop(0, n_steady // 6)
    def _(r):
        for u in range(6):
            full_step(1 + u, tr=1 + r * 6 + u)

    for t in range(1 + n_steady, nchunks - 2):
        full_step(t)

    t = nchunks - 2
    br, be = t % 2, t % 3
    gather_wait(br, be)
    mult(br, be)
    scatter_start(br, be)
    scatter_wait(1 - br, (t + 2) % 3)
    ed_wait((t + 1) % 3)
    adjust((t + 1) % 3)
    gather_start(1 - br, (t + 1) % 3)
    t = nchunks - 1
    br, be = t % 2, t % 3
    gather_wait(br, be)
    mult(br, be)
    scatter_start(br, be)
    scatter_wait(1 - br, (t + 2) % 3)
    scatter_wait(br, be)

    plsc.subcore_barrier()

    pltpu.sync_copy(accum.at[pl.ds(s * ROWS_PER_SUB, ROWS_PER_SUB), :],
                    p_hbm.at[c].at[pl.ds(s * ROWS_PER_SUB, ROWS_PER_SUB), :])


def _sc_compiler_params():
    cp = pltpu.CompilerParams()
    if "needs_layout_passes" in pltpu.CompilerParams.__dataclass_fields__:
        cp = dataclasses.replace(cp, needs_layout_passes=False)
    return cp


def _make_spmm(edge_split_cores, dcols):
    mesh = plsc.VectorSubcoreMesh(core_axis_name="c", subcore_axis_name="s")
    kern = functools.partial(_spmm_kernel_body, edge_split_cores, dcols)
    return pl.kernel(
        kern,
        compiler_params=_sc_compiler_params(),
        out_type=jax.ShapeDtypeStruct((NC, N_PAD, dcols), jnp.float32),
        mesh=mesh,
        scratch_types=[
            pltpu.VMEM((NSLOT, EDROWS, CHUNK), jnp.int32),
            pltpu.VMEM((2, CHUNK, dcols), jnp.float32),
            pltpu.VMEM((ZROWS, dcols), jnp.float32),
            pltpu.VMEM_SHARED((N_PAD, dcols), jnp.float32),
            pltpu.SemaphoreType.DMA((NSLOT,)),
            pltpu.SemaphoreType.DMA((2,)),
            pltpu.SemaphoreType.DMA((2,)),
        ],
    )


_spmm_l1 = _make_spmm(edge_split_cores=True, dcols=128)
_spmm_l2 = _make_spmm(edge_split_cores=False, dcols=128)

_ROWS_BLK = 400


def _mm1(P, W1, b1):
    def body(p_ref, w_ref, b_ref, o_ref):
        z = p_ref[0] + p_ref[1]
        h = jnp.dot(z, w_ref[...], preferred_element_type=jnp.float32)
        h = jnp.maximum(h + b_ref[...], 0.0)
        o_ref[0] = h[:, :128]
        o_ref[1] = h[:, 128:]

    return pl.pallas_call(
        body,
        grid=(N_NODES // _ROWS_BLK,),
        in_specs=[
            pl.BlockSpec((NC, _ROWS_BLK, 128), lambda i: (0, i, 0)),
            pl.BlockSpec((D_IN, D_HID), lambda i: (0, 0)),
            pl.BlockSpec((1, D_HID), lambda i: (0, 0)),
        ],
        out_specs=pl.BlockSpec((NC, _ROWS_BLK, 128), lambda i: (0, i, 0)),
        out_shape=jax.ShapeDtypeStruct((NC, N_NODES, 128), jnp.float32),
    )(P, W1, b1)


def _mm2(Z2, W2r, b2, Wfc, bfc):
    def body(z_ref, w2_ref, b2_ref, wfc_ref, bfc_ref, o_ref):
        h = jnp.dot(z_ref[0], w2_ref[0], preferred_element_type=jnp.float32)
        h = h + jnp.dot(z_ref[1], w2_ref[1], preferred_element_type=jnp.float32)
        h = jnp.maximum(h + b2_ref[...], 0.0)
        o = jnp.dot(h, wfc_ref[...], preferred_element_type=jnp.float32)
        o_ref[...] = o + bfc_ref[...]

    return pl.pallas_call(
        body,
        grid=(N_NODES // _ROWS_BLK,),
        in_specs=[
            pl.BlockSpec((NC, _ROWS_BLK, 128), lambda i: (0, i, 0)),
            pl.BlockSpec((NC, 128, D_HID), lambda i: (0, 0, 0)),
            pl.BlockSpec((1, D_HID), lambda i: (0, 0)),
            pl.BlockSpec((D_HID, D_OUT), lambda i: (0, 0)),
            pl.BlockSpec((1, D_OUT), lambda i: (0, 0)),
        ],
        out_specs=pl.BlockSpec((_ROWS_BLK, D_OUT), lambda i: (i, 0)),
        out_shape=jax.ShapeDtypeStruct((N_NODES, D_OUT), jnp.float32),
    )(Z2, W2r, b2, Wfc, bfc)


def kernel(x, edge_index, edge_weight, W1, b1, W2, b2, Wfc, bfc):
    src = edge_index[0]
    dst = edge_index[1]
    pad = E_PAD - N_EDGES
    src_p = jnp.concatenate([src, jnp.zeros((pad,), src.dtype)])
    dst_p = jnp.concatenate([dst, jnp.zeros((pad,), dst.dtype)])
    w_p = jnp.concatenate([edge_weight, jnp.zeros((pad,), edge_weight.dtype)])
    w32 = jax.lax.bitcast_convert_type(w_p, jnp.int32)
    edata = jnp.stack([src_p.reshape(-1, CHUNK), dst_p.reshape(-1, CHUNK),
                       w32.reshape(-1, CHUNK)], axis=1)
    edata = jnp.pad(edata, ((0, 0), (0, EDROWS - 3), (0, 0)))

    P = _spmm_l1(x, edata)
    h1 = _mm1(P, W1, b1.reshape(1, D_HID))
    Z2 = _spmm_l2(h1.reshape(NC * N_NODES, 128), edata)
    out = _mm2(Z2, W2.reshape(NC, 128, D_HID), b2.reshape(1, D_HID),
               Wfc, bfc.reshape(1, D_OUT))
    return out


# speedup vs baseline: 2.8977x; 2.8977x over previous
import dataclasses
import functools

import jax
import jax.numpy as jnp
from jax import lax
from jax.experimental import pallas as pl
from jax.experimental.pallas import tpu as pltpu
from jax.experimental.pallas import tpu_sc as plsc

N_NODES = 10000
N_EDGES = 320000
D_IN = 128
D_HID = 256
D_OUT = 128

NC = 2
NS = 16
LANES = 16

CHUNK = 64
N_PAD = 10240
E_PAD = 321536

ROWS_PER_SUB = N_PAD // NS
ZROWS = 16
NSLOT = 4


def _spmm_kernel_body(edge_split_cores, dcols, x_hbm, src_hbm, dst_hbm, w_hbm,
                      p_hbm, srcv, dstv, wv, rows, zbuf, accum,
                      esem, gsem, ssem):
    c = lax.axis_index("c")
    s = lax.axis_index("s")
    ngroups = dcols // LANES

    @pl.loop(0, ZROWS)
    def _(i):
        for g in range(ngroups):
            zbuf[i, pl.ds(g * LANES, LANES)] = jnp.zeros((LANES,), jnp.float32)

    @pl.loop(0, ROWS_PER_SUB // ZROWS)
    def _(j):
        pltpu.sync_copy(zbuf, accum.at[pl.ds(s * ROWS_PER_SUB + j * ZROWS, ZROWS), :])

    plsc.subcore_barrier()

    if edge_split_cores:
        wid = s * NC + c
        per_w = E_PAD // (NC * NS)
        row_off = None
    else:
        wid = s
        per_w = E_PAD // NS
        row_off = c * N_NODES
    nchunks = per_w // CHUNK
    ebase = wid * per_w

    def ed_start(t, b):
        off = ebase + t * CHUNK
        pltpu.async_copy(src_hbm.at[pl.ds(off, CHUNK)], srcv.at[b], esem.at[b])
        pltpu.async_copy(dst_hbm.at[pl.ds(off, CHUNK)], dstv.at[b], esem.at[b])
        pltpu.async_copy(w_hbm.at[pl.ds(off, CHUNK)], wv.at[b], esem.at[b])

    def ed_wait(b):
        pltpu.make_async_copy(src_hbm.at[pl.ds(0, CHUNK)], srcv.at[b],
                              esem.at[b]).wait()
        pltpu.make_async_copy(dst_hbm.at[pl.ds(0, CHUNK)], dstv.at[b],
                              esem.at[b]).wait()
        pltpu.make_async_copy(w_hbm.at[pl.ds(0, CHUNK)], wv.at[b],
                              esem.at[b]).wait()

    def adjust(b):
        if row_off is not None:
            for q in range(CHUNK // LANES):
                sl = (b, pl.ds(q * LANES, LANES))
                srcv[sl] = srcv[sl] + row_off

    def gather_start(b):
        pltpu.async_copy(x_hbm.at[srcv.at[b]], rows.at[b], gsem.at[b])

    def gather_wait(b):
        pltpu.make_async_copy(x_hbm.at[srcv.at[b]], rows.at[b],
                              gsem.at[b]).wait()

    def mult(b):
        @pl.loop(0, CHUNK // LANES)
        def _(q):
            wreg = wv[b, pl.ds(q * LANES, LANES)]
            for j in range(LANES):
                wt = wreg[j]
                row = q * LANES + j
                for g in range(ngroups):
                    sl = (b, row, pl.ds(g * LANES, LANES))
                    rows[sl] = rows[sl] * wt

    def scatter_start(b):
        pltpu.async_copy(rows.at[b], accum.at[dstv.at[b]], ssem.at[b],
                         add=True)

    def scatter_wait(b):
        pltpu.make_async_copy(rows.at[b], accum.at[dstv.at[b]],
                              ssem.at[b]).wait()

    def full_step(t, tr=None, do_swait=True, do_gather=True, do_ed=True):
        ti = t if tr is None else tr
        b, b2, b3 = t % NSLOT, (t + 2) % NSLOT, (t + 3) % NSLOT
        gather_wait(b)
        if do_swait:
            scatter_wait(b2)
        if do_gather:
            ed_wait(b2)
            adjust(b2)
            gather_start(b2)
        if do_ed:
            ed_start(ti + 3, b3)
        mult(b)
        scatter_start(b)

    ed_start(0, 0)
    ed_start(1, 1)
    ed_start(2, 2)
    ed_wait(0)
    adjust(0)
    gather_start(0)
    ed_wait(1)
    adjust(1)
    gather_start(1)
    full_step(0, do_swait=False)
    full_step(1, do_swait=False)

    n_steady = ((nchunks - 5) // NSLOT) * NSLOT

    @pl.loop(0, n_steady // NSLOT)
    def _(r):
        for u in range(NSLOT):
            full_step(2 + u, tr=2 + r * NSLOT + u)

    for t in range(2 + n_steady, nchunks - 3):
        full_step(t)
    full_step(nchunks - 3, do_ed=False)
    full_step(nchunks - 2, do_ed=False, do_gather=False)
    full_step(nchunks - 1, do_ed=False, do_gather=False)
    scatter_wait((nchunks - 2) % NSLOT)
    scatter_wait((nchunks - 1) % NSLOT)

    plsc.subcore_barrier()

    pltpu.sync_copy(accum.at[pl.ds(s * ROWS_PER_SUB, ROWS_PER_SUB), :],
                    p_hbm.at[c].at[pl.ds(s * ROWS_PER_SUB, ROWS_PER_SUB), :])


def _sc_compiler_params():
    cp = pltpu.CompilerParams()
    if "needs_layout_passes" in pltpu.CompilerParams.__dataclass_fields__:
        cp = dataclasses.replace(cp, needs_layout_passes=False)
    return cp


def _make_spmm(edge_split_cores, dcols):
    mesh = plsc.VectorSubcoreMesh(core_axis_name="c", subcore_axis_name="s")
    kern = functools.partial(_spmm_kernel_body, edge_split_cores, dcols)
    return pl.kernel(
        kern,
        compiler_params=_sc_compiler_params(),
        out_type=jax.ShapeDtypeStruct((NC, N_PAD, dcols), jnp.float32),
        mesh=mesh,
        scratch_types=[
            pltpu.VMEM((NSLOT, CHUNK), jnp.int32),
            pltpu.VMEM((NSLOT, CHUNK), jnp.int32),
            pltpu.VMEM((NSLOT, CHUNK), jnp.float32),
            pltpu.VMEM((NSLOT, CHUNK, dcols), jnp.float32),
            pltpu.VMEM((ZROWS, dcols), jnp.float32),
            pltpu.VMEM_SHARED((N_PAD, dcols), jnp.float32),
            pltpu.SemaphoreType.DMA((NSLOT,)),
            pltpu.SemaphoreType.DMA((NSLOT,)),
            pltpu.SemaphoreType.DMA((NSLOT,)),
        ],
    )


_spmm_l1 = _make_spmm(edge_split_cores=True, dcols=128)
_spmm_l2 = _make_spmm(edge_split_cores=False, dcols=128)

_ROWS_BLK = 400


def _mm1(P, W1, b1):
    def body(p_ref, w_ref, b_ref, o_ref):
        z = p_ref[0] + p_ref[1]
        h = jnp.dot(z, w_ref[...], preferred_element_type=jnp.float32)
        h = jnp.maximum(h + b_ref[...], 0.0)
        o_ref[0] = h[:, :128]
        o_ref[1] = h[:, 128:]

    return pl.pallas_call(
        body,
        grid=(N_NODES // _ROWS_BLK,),
        in_specs=[
            pl.BlockSpec((NC, _ROWS_BLK, 128), lambda i: (0, i, 0)),
            pl.BlockSpec((D_IN, D_HID), lambda i: (0, 0)),
            pl.BlockSpec((1, D_HID), lambda i: (0, 0)),
        ],
        out_specs=pl.BlockSpec((NC, _ROWS_BLK, 128), lambda i: (0, i, 0)),
        out_shape=jax.ShapeDtypeStruct((NC, N_NODES, 128), jnp.float32),
    )(P, W1, b1)


def _mm2(Z2, W2r, b2, Wfc, bfc):
    def body(z_ref, w2_ref, b2_ref, wfc_ref, bfc_ref, o_ref):
        h = jnp.dot(z_ref[0], w2_ref[0], preferred_element_type=jnp.float32)
        h = h + jnp.dot(z_ref[1], w2_ref[1], preferred_element_type=jnp.float32)
        h = jnp.maximum(h + b2_ref[...], 0.0)
        o = jnp.dot(h, wfc_ref[...], preferred_element_type=jnp.float32)
        o_ref[...] = o + bfc_ref[...]

    return pl.pallas_call(
        body,
        grid=(N_NODES // _ROWS_BLK,),
        in_specs=[
            pl.BlockSpec((NC, _ROWS_BLK, 128), lambda i: (0, i, 0)),
            pl.BlockSpec((NC, 128, D_HID), lambda i: (0, 0, 0)),
            pl.BlockSpec((1, D_HID), lambda i: (0, 0)),
            pl.BlockSpec((D_HID, D_OUT), lambda i: (0, 0)),
            pl.BlockSpec((1, D_OUT), lambda i: (0, 0)),
        ],
        out_specs=pl.BlockSpec((_ROWS_BLK, D_OUT), lambda i: (i, 0)),
        out_shape=jax.ShapeDtypeStruct((N_NODES, D_OUT), jnp.float32),
    )(Z2, W2r, b2, Wfc, bfc)


def kernel(x, edge_index, edge_weight, W1, b1, W2, b2, Wfc, bfc):
    src = edge_index[0]
    dst = edge_index[1]
    pad = E_PAD - N_EDGES
    src_p = jnp.concatenate([src, jnp.zeros((pad,), src.dtype)])
    dst_p = jnp.concatenate([dst, jnp.zeros((pad,), dst.dtype)])
    w_p = jnp.concatenate([edge_weight, jnp.zeros((pad,), edge_weight.dtype)])

    P = _spmm_l1(x, src_p, dst_p, w_p)
    h1 = _mm1(P, W1, b1.reshape(1, D_HID))
    Z2 = _spmm_l2(h1.reshape(NC * N_NODES, 128), src_p, dst_p, w_p)
    out = _mm2(Z2, W2.reshape(NC, 128, D_HID), b2.reshape(1, D_HID),
               Wfc, bfc.reshape(1, D_OUT))
    return out
